# baseline (speedup 1.0000x reference)
import functools

import jax
import jax.numpy as jnp
from jax import lax
from jax.experimental import pallas as pl
from jax.experimental.pallas import tpu as pltpu
from jax.experimental.pallas import tpu_sc as plsc

B = 4096
L = 200
D = 64
NW = 32
BPW = B // NW
NC = BPW // 16
NBUF = 2


def _sc_body(textT_hbm, len_hbm, table2_hbm, embsT_hbm, seqT_hbm,
             stage, acc, lens_v, idx2_0, idx2_1, par_0, par_1,
             pair_0, pair_1,
             build_0, build_1, gsem_0, gsem_1, osem_0, osem_1, isem):
    idx2_bufs = (idx2_0, idx2_1)
    par_bufs = (par_0, par_1)
    pair_bufs = (pair_0, pair_1)
    build_bufs = (build_0, build_1)
    gsems = (gsem_0, gsem_1)
    osems = (osem_0, osem_1)

    c = lax.axis_index("c")
    s = lax.axis_index("s")
    wid = s * 2 + c
    base = pl.multiple_of(wid * BPW, BPW)

    iotas = [lax.iota(jnp.int32, 16) + 16 * cc for cc in range(NC)]

    pltpu.sync_copy(len_hbm.at[pl.ds(base, BPW)], lens_v)
    pltpu.sync_copy(textT_hbm.at[pl.ds(0, 8), pl.ds(base, BPW)],
                    stage.at[0])
    pltpu.async_copy(textT_hbm.at[pl.ds(8, 8), pl.ds(base, BPW)],
                     stage.at[1], isem)

    def start_gather(l, b):
        ph = (l >> 3) & 1
        row = l & 7
        for cc in range(NC):
            v = stage[ph, row, pl.ds(16 * cc, 16)]
            idx2_bufs[b][pl.ds(16 * cc, 16)] = lax.shift_right_logical(v, 1)
            par_bufs[b][pl.ds(16 * cc, 16)] = v & 1
        pltpu.async_copy(table2_hbm.at[idx2_bufs[b]], pair_bufs[b], gsems[b])

    def drain_gather(b):
        pltpu.make_async_copy(table2_hbm.at[idx2_bufs[b]], pair_bufs[b],
                              gsems[b]).wait()

    def zero_d(i, carry):
        z = jnp.zeros((16,), jnp.float32)
        for cc in range(NC):
            acc[i, pl.ds(16 * cc, 16)] = z
        return carry
    lax.fori_loop(0, D, zero_d, 0)

    for b in range(NBUF):
        start_gather(b, b)

    lens16 = tuple(lens_v[pl.ds(16 * cc, 16)] for cc in range(NC))

    def outer(l2, carry):
        for b in range(NBUF):
            l = l2 * NBUF + b

            @pl.when(jnp.logical_and((l & 7) == 6, l + 2 < L))
            def _():
                pltpu.make_async_copy(
                    textT_hbm.at[pl.ds(0, 8), pl.ds(base, BPW)],
                    stage.at[((l + 2) >> 3) & 1], isem).wait()

            @pl.when(jnp.logical_and((l & 7) == 6, l + 10 < L))
            def _():
                pltpu.async_copy(
                    textT_hbm.at[pl.ds(pl.multiple_of(l + 10, 8), 8),
                                 pl.ds(base, BPW)],
                    stage.at[((l + 10) >> 3) & 1], isem)

            drain_gather(b)

            @pl.when(l >= NBUF)
            def _():
                pltpu.make_async_copy(
                    build_bufs[b],
                    embsT_hbm.at[l - NBUF, :, pl.ds(base, BPW)],
                    osems[b]).wait()

            lsplat = jnp.full((16,), l, jnp.int32)
            for cc in range(NC):
                par16 = par_bufs[b][pl.ds(16 * cc, 16)]
                colbase = par16 * 64
                mask16 = lsplat < carry[cc]
                rows16 = iotas[cc]
                dsplat = jnp.zeros((16,), jnp.int32)
                for d in range(D):
                    cols = colbase + d
                    v = plsc.load_gather(pair_bufs[b], [rows16, cols])
                    build_bufs[b][d, pl.ds(16 * cc, 16)] = v
                    plsc.addupdate_scatter(
                        acc, [dsplat + d, rows16], v, mask=mask16)

            pltpu.async_copy(build_bufs[b],
                             embsT_hbm.at[l, :, pl.ds(base, BPW)], osems[b])

            @pl.when(l + NBUF < L)
            def _():
                start_gather(l + NBUF, b)
        return carry

    lax.fori_loop(0, L // NBUF, outer, lens16)

    for b in range(NBUF):
        pltpu.make_async_copy(build_bufs[b],
                              embsT_hbm.at[L - NBUF + b, :,
                                           pl.ds(base, BPW)],
                              osems[b]).wait()

    lenf16 = tuple(carry.astype(jnp.float32) for carry in lens16)

    def div_d(i, carry):
        for cc in range(NC):
            acc[i, pl.ds(16 * cc, 16)] = (
                acc[i, pl.ds(16 * cc, 16)] / lenf16[cc])
        return carry
    lax.fori_loop(0, D, div_d, 0)
    pltpu.sync_copy(acc, seqT_hbm.at[:, pl.ds(base, BPW)])


@functools.partial(jax.jit, static_argnames=())
def _run(textT, text_len, table2):
    mesh = plsc.VectorSubcoreMesh(core_axis_name="c", subcore_axis_name="s")
    k = pl.kernel(
        _sc_body,
        mesh=mesh,
        out_type=[
            jax.ShapeDtypeStruct((L, D, B), jnp.float32),
            jax.ShapeDtypeStruct((D, B), jnp.float32),
        ],
        scratch_types=(
            [
                pltpu.VMEM((2, 8, BPW), jnp.int32),
                pltpu.VMEM((D, BPW), jnp.float32),
                pltpu.VMEM((BPW,), jnp.int32),
            ]
            + [pltpu.VMEM((BPW,), jnp.int32) for _ in range(NBUF)]
            + [pltpu.VMEM((BPW,), jnp.int32) for _ in range(NBUF)]
            + [pltpu.VMEM((BPW, 128), jnp.float32) for _ in range(NBUF)]
            + [pltpu.VMEM((D, BPW), jnp.float32) for _ in range(NBUF)]
            + [pltpu.SemaphoreType.DMA for _ in range(2 * NBUF + 1)]
        ),
        compiler_params=pltpu.CompilerParams(needs_layout_passes=False),
    )
    return k(textT, text_len, table2)


def kernel(text, text_len, table):
    textT = jnp.transpose(text.astype(jnp.int32))
    table2 = table.reshape(500000, 128)
    embsT, seqT = _run(textT, text_len.astype(jnp.int32), table2)
    embs = jnp.transpose(embsT, (2, 0, 1))
    seq = jnp.transpose(seqT)
    return embs, seq


# speedup vs baseline: 1.7328x; 1.7328x over previous
import functools

import jax
import jax.numpy as jnp
from jax import lax
from jax.experimental import pallas as pl
from jax.experimental.pallas import tpu as pltpu
from jax.experimental.pallas import tpu_sc as plsc

B = 4096
L = 200
D = 64
NW = 32
BPW = B // NW
LG = D // 16
LH = L // 2


NBUF = 4


def _sc_body(text_hbm, len_hbm, table_hbm, embs_hbm, seq_hbm,
             idx_all, seq_acc, lens_v, *bufs_and_sems):
    rows_bufs = bufs_and_sems[:NBUF]
    gsems = bufs_and_sems[NBUF:2 * NBUF]
    osems = bufs_and_sems[2 * NBUF:3 * NBUF]
    c = lax.axis_index("c")
    s = lax.axis_index("s")
    wid = s * 2 + c
    base = wid * BPW

    pltpu.sync_copy(text_hbm.at[pl.ds(base, BPW)], idx_all)
    pltpu.sync_copy(len_hbm.at[pl.ds(base, BPW)], lens_v.at[pl.ds(0, BPW)])

    def start_gather(r, b):
        pltpu.async_copy(table_hbm.at[idx_all.at[r, pl.ds(0, 128)]],
                         rows_bufs[b].at[pl.ds(0, 128)], gsems[b])
        pltpu.async_copy(table_hbm.at[idx_all.at[r, pl.ds(128, 72)]],
                         rows_bufs[b].at[pl.ds(128, 72)], gsems[b])

    def drain_gather(r, b):
        pltpu.make_async_copy(table_hbm.at[idx_all.at[r, pl.ds(0, 128)]],
                              rows_bufs[b].at[pl.ds(0, 128)], gsems[b]).wait()
        pltpu.make_async_copy(table_hbm.at[idx_all.at[r, pl.ds(128, 72)]],
                              rows_bufs[b].at[pl.ds(128, 72)], gsems[b]).wait()

    for b in range(NBUF):
        start_gather(b, b)

    def outer(g, carry):
        r0 = g * NBUF
        for b in range(NBUF):
            r = r0 + b
            drain_gather(r, b)
            out_cp = pltpu.make_async_copy(
                rows_bufs[b], embs_hbm.at[base + r], osems[b])
            out_cp.start()

            ln = lens_v[pl.ds(r, 16)][0]

            def acc_body(i, acc, _b=b):
                return tuple(acc[q] + rows_bufs[_b][i, pl.ds(q * 16, 16)]
                             for q in range(LG))

            acc = lax.fori_loop(
                0, ln, acc_body,
                tuple(jnp.zeros((16,), jnp.float32) for _ in range(LG)))
            lf_v = jnp.full((16,), ln.astype(jnp.float32))
            for q in range(LG):
                seq_acc[r, pl.ds(q * 16, 16)] = acc[q] / lf_v

            out_cp.wait()

            @pl.when(r + NBUF < BPW)
            def _():
                start_gather(r + NBUF, b)
        return carry

    lax.fori_loop(0, BPW // NBUF, outer, 0)
    pltpu.sync_copy(seq_acc, seq_hbm.at[pl.ds(base, BPW)])


@functools.partial(jax.jit, static_argnames=())
def _run(text_r, text_len, table):
    mesh = plsc.VectorSubcoreMesh(core_axis_name="c", subcore_axis_name="s")
    k = pl.kernel(
        _sc_body,
        mesh=mesh,
        out_type=[
            jax.ShapeDtypeStruct((B, L, D), jnp.float32),
            jax.ShapeDtypeStruct((B, D), jnp.float32),
        ],
        scratch_types=(
            [
                pltpu.VMEM((BPW, L), jnp.int32),
                pltpu.VMEM((BPW, D), jnp.float32),
                pltpu.VMEM((BPW + 16,), jnp.int32),
            ]
            + [pltpu.VMEM((L, D), jnp.float32) for _ in range(NBUF)]
            + [pltpu.SemaphoreType.DMA for _ in range(2 * NBUF)]
        ),
        compiler_params=pltpu.CompilerParams(use_tc_tiling_on_sc=False),
    )
    return k(text_r, text_len, table)


def kernel(text, text_len, table):
    embs, seq = _run(text.astype(jnp.int32), text_len.astype(jnp.int32), table)
    return embs, seq
